# baseline (speedup 1.0000x reference)
import functools

import jax
import jax.numpy as jnp
from jax import lax
from jax.experimental import pallas as pl
from jax.experimental.pallas import tpu as pltpu
from jax.experimental.pallas import tpu_sc as plsc

B, Q, N = 64, 32, 4096
R = B * Q
NC, NS, L = 2, 16, 16
NW = NC * NS
RB = 8
NCT = N // 128
GSZ = RB * N
UNROLL = 4

RS = 256
RT = R - RS
SC_ROWS_PER_W = RS // NW
SC_NBATCH = SC_ROWS_PER_W // RB
TC_BLK = 128

_IBIG = 0x7FFFFFFF


def _group_scan(buf):
    best0 = tuple(jnp.full((L,), jnp.inf, jnp.float32) for _ in range(RB))
    bidx0 = tuple(jnp.zeros((L,), jnp.int32) for _ in range(RB))
    tv0 = jnp.zeros((L,), jnp.int32)

    def body(s, carry):
        best, bidx, tv = carry
        best, bidx = list(best), list(bidx)
        base = (s // 2) * (RB * 128) + (s % 2) * (UNROLL * L)
        for ju in range(UNROLL):
            tvu = tv + ju
            for r in range(RB):
                v = buf[pl.ds(base + r * 128 + ju * L, L)]
                m = v < best[r]
                best[r] = jnp.where(m, v, best[r])
                bidx[r] = jnp.where(m, tvu, bidx[r])
        return tuple(best), tuple(bidx), tv + UNROLL

    best, bidx, _ = lax.fori_loop(0, NCT * 2, body, (best0, bidx0, tv0))
    return best, bidx


def _finalize(best, bidx, lanes, parity, res):
    for r in range(RB):
        iabs = bidx[r] * L + lanes
        vmin = jnp.min(best[r])
        cand = jnp.where(best[r] == vmin, iabs, _IBIG)
        imin = jnp.min(cand)
        res = jnp.where(lanes == parity * RB + r, imin, res)
    return res


def _tec_body(x_hbm, dep_hbm, out_hbm, buf_a, buf_b, out_v, sem_a, sem_b):
    del dep_hbm
    wid = lax.axis_index("s") * NC + lax.axis_index("c")
    row0 = RT + wid * SC_ROWS_PER_W
    bufs = (buf_a, buf_b)
    sems = (sem_a, sem_b)
    lanes = jnp.arange(L, dtype=jnp.int32)
    copies = {}

    def start(i):
        src = x_hbm.at[pl.ds((row0 + i * RB) * N, GSZ)]
        copies[i] = pltpu.async_copy(src, bufs[i % 2], sems[i % 2])

    start(0)
    res = jnp.zeros((L,), jnp.int32)
    for i in range(SC_NBATCH):
        if i + 1 < SC_NBATCH:
            start(i + 1)
        copies[i].wait()
        best, bidx = _group_scan(bufs[i % 2])
        res = _finalize(best, bidx, lanes, i % 2, res)
        if i % 2 == 1 or i == SC_NBATCH - 1:
            out_v[pl.ds((i // 2) * L, L)] = res
    nout = SC_ROWS_PER_W
    pltpu.sync_copy(out_v.at[pl.ds(0, nout)],
                    out_hbm.at[pl.ds(wid * nout, nout)])


@functools.cache
def _build_sc():
    nvout = max(L, SC_ROWS_PER_W)
    return pl.kernel(
        _tec_body,
        out_type=jax.ShapeDtypeStruct((RS,), jnp.int32),
        mesh=plsc.VectorSubcoreMesh(
            core_axis_name="c", subcore_axis_name="s",
            num_cores=NC, num_subcores=NS),
        compiler_params=pltpu.CompilerParams(
            use_tc_tiling_on_sc=False, needs_layout_passes=False,
            skip_device_barrier=True),
        scratch_types=[
            pltpu.VMEM((GSZ,), jnp.float32),
            pltpu.VMEM((GSZ,), jnp.float32),
            pltpu.VMEM((nvout,), jnp.int32),
            pltpu.SemaphoreType.DMA,
            pltpu.SemaphoreType.DMA,
        ],
    )


def _tc_body(x_ref, o_ref):
    xb = x_ref[...]
    o_ref[0, 0, :] = jnp.argmin(xb, axis=1).astype(jnp.int32)


TCA_BLOCKS = 6


@functools.cache
def _build_tc(nblk, off):
    return pl.pallas_call(
        _tc_body,
        grid=(nblk,),
        in_specs=[pl.BlockSpec((TC_BLK, N), lambda i: (i + off, 0))],
        out_specs=pl.BlockSpec((1, 1, TC_BLK), lambda i: (i, 0, 0)),
        out_shape=jax.ShapeDtypeStruct((nblk, 1, TC_BLK), jnp.int32),
    )


def kernel(x):
    x2 = x.reshape(R, N)
    y = (x.reshape(B, Q // RB, RB, NCT, 128)
         .transpose(0, 1, 3, 2, 4)
         .reshape(R * N))
    grid = RT // TC_BLK
    out_a = _build_tc(TCA_BLOCKS, 0)(x2)
    dep = lax.slice(out_a, (0, 0, 0), (1, 1, 8)).reshape(8)
    out_sc = _build_sc()(y, dep)
    out_b = _build_tc(grid - TCA_BLOCKS, TCA_BLOCKS)(x2)
    out = jnp.concatenate(
        [out_a.reshape(TCA_BLOCKS * TC_BLK),
         out_b.reshape(RT - TCA_BLOCKS * TC_BLK),
         out_sc])
    return out.reshape(B, Q).astype(jnp.int64)


# speedup vs baseline: 1.1067x; 1.1067x over previous
import functools

import jax
import jax.numpy as jnp
from jax import lax
from jax.experimental import pallas as pl
from jax.experimental.pallas import tpu as pltpu
from jax.experimental.pallas import tpu_sc as plsc

B, Q, N = 64, 32, 4096
R = B * Q
NC, NS, L = 2, 16, 16
NW = NC * NS
RB = 8
NCT = N // 128
GSZ = RB * N
UNROLL = 4

RS = 512
RT = R - RS
SC_ROWS_PER_W = RS // NW
SC_NBATCH = SC_ROWS_PER_W // RB
TC_BLK = 256

_IBIG = 0x7FFFFFFF


def _group_scan(buf):
    best0 = tuple(jnp.full((L,), jnp.inf, jnp.float32) for _ in range(RB))
    bidx0 = tuple(jnp.zeros((L,), jnp.int32) for _ in range(RB))
    tv0 = jnp.zeros((L,), jnp.int32)

    def body(s, carry):
        best, bidx, tv = carry
        best, bidx = list(best), list(bidx)
        base = (s // 2) * (RB * 128) + (s % 2) * (UNROLL * L)
        for ju in range(UNROLL):
            tvu = tv + ju
            for r in range(RB):
                v = buf[pl.ds(base + r * 128 + ju * L, L)]
                m = v < best[r]
                best[r] = jnp.where(m, v, best[r])
                bidx[r] = jnp.where(m, tvu, bidx[r])
        return tuple(best), tuple(bidx), tv + UNROLL

    best, bidx, _ = lax.fori_loop(0, NCT * 2, body, (best0, bidx0, tv0))
    return best, bidx


def _finalize(best, bidx, lanes, parity, res):
    for r in range(RB):
        iabs = bidx[r] * L + lanes
        vmin = jnp.min(best[r])
        cand = jnp.where(best[r] == vmin, iabs, _IBIG)
        imin = jnp.min(cand)
        res = jnp.where(lanes == parity * RB + r, imin, res)
    return res


def _tec_body(x_hbm, out_hbm, buf_a, buf_b, out_v, sem_a, sem_b):
    wid = lax.axis_index("s") * NC + lax.axis_index("c")
    row0 = RT + wid * SC_ROWS_PER_W
    bufs = (buf_a, buf_b)
    sems = (sem_a, sem_b)
    lanes = jnp.arange(L, dtype=jnp.int32)
    copies = {}

    def start(i):
        src = x_hbm.at[pl.ds((row0 + i * RB) * N, GSZ)]
        copies[i] = pltpu.async_copy(src, bufs[i % 2], sems[i % 2])

    start(0)
    res = jnp.zeros((L,), jnp.int32)
    for i in range(SC_NBATCH):
        if i + 1 < SC_NBATCH:
            start(i + 1)
        copies[i].wait()
        best, bidx = _group_scan(bufs[i % 2])
        res = _finalize(best, bidx, lanes, i % 2, res)
        if i % 2 == 1 or i == SC_NBATCH - 1:
            out_v[pl.ds((i // 2) * L, L)] = res
    nout = SC_ROWS_PER_W
    pltpu.sync_copy(out_v.at[pl.ds(0, nout)],
                    out_hbm.at[pl.ds(wid * nout, nout)])


@functools.cache
def _build_sc():
    nvout = max(L, SC_ROWS_PER_W)
    return pl.kernel(
        _tec_body,
        out_type=jax.ShapeDtypeStruct((RS,), jnp.int32),
        mesh=plsc.VectorSubcoreMesh(
            core_axis_name="c", subcore_axis_name="s",
            num_cores=NC, num_subcores=NS),
        compiler_params=pltpu.CompilerParams(
            use_tc_tiling_on_sc=False, needs_layout_passes=False,
            skip_device_barrier=True),
        scratch_types=[
            pltpu.VMEM((GSZ,), jnp.float32),
            pltpu.VMEM((GSZ,), jnp.float32),
            pltpu.VMEM((nvout,), jnp.int32),
            pltpu.SemaphoreType.DMA,
            pltpu.SemaphoreType.DMA,
        ],
    )


def _tc_body(x_ref, o_ref):
    xb = x_ref[...]
    o_ref[0, 0, :] = jnp.argmin(xb, axis=1).astype(jnp.int32)


@functools.cache
def _build_tc():
    grid = RT // TC_BLK
    return pl.pallas_call(
        _tc_body,
        grid=(grid,),
        in_specs=[pl.BlockSpec((TC_BLK, N), lambda i: (i, 0))],
        out_specs=pl.BlockSpec((1, 1, TC_BLK), lambda i: (i, 0, 0)),
        out_shape=jax.ShapeDtypeStruct((grid, 1, TC_BLK), jnp.int32),
    )


def kernel(x):
    x2 = x.reshape(R, N)
    y = (x.reshape(B, Q // RB, RB, NCT, 128)
         .transpose(0, 1, 3, 2, 4)
         .reshape(R * N))
    out_sc = _build_sc()(y)
    out_tc = _build_tc()(x2).reshape(RT)
    out = jnp.concatenate([out_tc, out_sc])
    return out.reshape(B, Q).astype(jnp.int64)
